# baseline (speedup 1.0000x reference)
import functools

import jax
import jax.numpy as jnp
import numpy as np
from jax import lax
from jax.experimental import pallas as pl
from jax.experimental.pallas import tpu as pltpu
from jax.experimental.pallas import tpu_sc as plsc

K = 3
DIM_FEATURE = 32
B = 4096
Hb = 15
Wb = 15
C = 2
H = Hb - K + 1
P = H * H
NW = 32
BOARDS_PER_W = B // NW
CHUNK = 128
QMAX = 127.0 / 128.0


def _build_unfold_matrix():
    m = np.zeros((C * Hb * Wb, P), dtype=np.float32)
    for c in range(C):
        for di in range(K):
            for dj in range(K):
                w = float(2 ** (c * 9 + di * 3 + dj))
                for i in range(H):
                    for j in range(H):
                        m[c * Hb * Wb + (i + di) * Wb + (j + dj), i * H + j] += w
    return m


_M_CONST = _build_unfold_matrix()
_ZERO_ROWS = np.zeros((CHUNK, DIM_FEATURE), dtype=np.int16)


def _index_body(m_ref, x_ref, o_ref):
    acc = jax.lax.dot_general(
        m_ref[...], x_ref[...], (((0,), (1,)), ((), ())),
        preferred_element_type=jnp.float32)
    o_ref[...] = acc.astype(jnp.int32)


def _compute_indices(board_flat, m):
    blk = 512
    return pl.pallas_call(
        _index_body,
        grid=(B // blk,),
        in_specs=[
            pl.BlockSpec((C * Hb * Wb, P), lambda i: (0, 0)),
            pl.BlockSpec((blk, C * Hb * Wb), lambda i: (i, 0)),
        ],
        out_specs=pl.BlockSpec((P, blk), lambda i: (0, i)),
        out_shape=jax.ShapeDtypeStruct((P, B), jnp.int32),
    )(m, board_flat)


def _quant_body(t_ref, o_ref):
    x = jnp.clip(t_ref[...], -1.0, QMAX)
    o_ref[...] = jnp.round(x * 128.0).astype(jnp.int16)


def _quantize_table(table):
    v, d = table.shape
    blk = 8192
    return pl.pallas_call(
        _quant_body,
        grid=(v // blk,),
        in_specs=[pl.BlockSpec((blk, d), lambda i: (i, 0))],
        out_specs=pl.BlockSpec((blk, d), lambda i: (i, 0)),
        out_shape=jax.ShapeDtypeStruct((v, d), jnp.int16),
    )(table)


def _sc_gather_sum(qtable, idxpm, zrows):
    mesh = plsc.VectorSubcoreMesh(core_axis_name="c", subcore_axis_name="s")

    @functools.partial(
        pl.kernel,
        out_type=jax.ShapeDtypeStruct((B, DIM_FEATURE), jnp.int16),
        mesh=mesh,
        compiler_params=pltpu.CompilerParams(use_tc_tiling_on_sc=False),
        scratch_types=[
            pltpu.VMEM((P, CHUNK), jnp.int32),
            pltpu.VMEM((CHUNK,), jnp.int32),
            pltpu.VMEM((4, CHUNK, DIM_FEATURE), jnp.int16),
            pltpu.VMEM_SHARED((B // 2, DIM_FEATURE), jnp.int16),
            pltpu.SemaphoreType.DMA,
            pltpu.SemaphoreType.DMA,
            pltpu.SemaphoreType.DMA,
            pltpu.SemaphoreType.DMA,
            pltpu.SemaphoreType.DMA,
            pltpu.SemaphoreType.DMA,
            pltpu.SemaphoreType.DMA,
            pltpu.SemaphoreType.DMA,
        ],
    )
    def sck(qtab_hbm, idx_hbm, zero_hbm, out_hbm,
            idx_v, bidx_v, rows_v, acc,
            g0, g1, g2, g3, s0, s1, s2, s3):
        c = lax.axis_index("c")
        s = lax.axis_index("s")
        wid = c * 16 + s
        pltpu.sync_copy(idx_hbm.at[:, pl.ds(wid * CHUNK, CHUNK)], idx_v)
        for k in range(CHUNK // 16):
            bidx_v[pl.ds(k * 16, 16)] = (
                lax.iota(jnp.int32, 16) + (s * CHUNK + k * 16))
        pltpu.sync_copy(zero_hbm, acc.at[pl.ds(s * CHUNK, CHUNK)])

        gs = (g0, g1, g2, g3)
        ss = (s0, s1, s2, s3)

        def gissue(j, b):
            pltpu.async_copy(qtab_hbm.at[idx_v.at[j]], rows_v.at[b], gs[b])

        def gwait(j, b):
            pltpu.make_async_copy(qtab_hbm.at[idx_v.at[j]],
                                  rows_v.at[b], gs[b]).wait()

        def sissue(b):
            pltpu.async_copy(rows_v.at[b], acc.at[bidx_v], ss[b], add=True)

        def swait(b):
            pltpu.make_async_copy(rows_v.at[b], acc.at[bidx_v], ss[b]).wait()

        for b in range(4):
            gissue(b, b)

        def body(t, carry):
            j = 4 * t
            for b in range(4):
                gwait(j + b, b)
                sissue(b)
            for b in range(4):
                swait(b)
                gissue(j + 4 + b, b)
            return carry

        lax.fori_loop(0, (P - 5) // 4, body, 0)
        for b in range(4):
            gwait(164 + b, b)
            sissue(b)
        swait(0)
        gissue(P - 1, 0)
        for b in range(1, 4):
            swait(b)
        gwait(P - 1, 0)
        sissue(0)
        swait(0)
        pltpu.sync_copy(acc.at[pl.ds(s * CHUNK, CHUNK)],
                        out_hbm.at[pl.ds(wid * CHUNK, CHUNK)])

    return sck(qtable, idxpm, zrows)


def _mlp_body(f_ref, w1_ref, b1_ref, w2_ref, b2_ref, w3_ref, b3_ref, o_ref):
    def wq(w):
        return jnp.clip(jnp.round(w * 128.0), -128.0, 127.0) * (1.0 / 128.0)

    def bq(b):
        return jnp.round(b * 16384.0) * (1.0 / 16384.0)

    v = f_ref[...].astype(jnp.float32) * (1.0 / 128.0)
    v = jnp.clip(v, -1.0, QMAX)
    v = jax.lax.dot_general(v, wq(w1_ref[...]), (((1,), (1,)), ((), ())),
                            preferred_element_type=jnp.float32) + bq(b1_ref[...])
    v = jnp.clip(v, 0.0, QMAX)
    v = jax.lax.dot_general(v, wq(w2_ref[...]), (((1,), (1,)), ((), ())),
                            preferred_element_type=jnp.float32) + bq(b2_ref[...])
    v = jnp.clip(v, 0.0, QMAX)
    v = jax.lax.dot_general(v, wq(w3_ref[...]), (((1,), (1,)), ((), ())),
                            preferred_element_type=jnp.float32) + bq(b3_ref[...])
    o_ref[...] = v


def _mlp(feature, W1, b1, W2, b2, W3, b3):
    return pl.pallas_call(
        _mlp_body,
        out_shape=jax.ShapeDtypeStruct((B, 3), jnp.float32),
    )(feature, W1, b1.reshape(1, -1), W2, b2.reshape(1, -1),
      W3, b3.reshape(1, -1))


def kernel(board_input, hash_features, W1, b1, W2, b2, W3, b3):
    board_flat = board_input.reshape(B, C * Hb * Wb)
    idx = _compute_indices(board_flat, jnp.asarray(_M_CONST))
    qtab = _quantize_table(hash_features)
    feature = _sc_gather_sum(qtab, idx, jnp.asarray(_ZERO_ROWS))
    value = _mlp(feature, W1, b1, W2, b2, W3, b3)
    policy = jnp.zeros((B, Hb, Wb), dtype=jnp.float32)
    return (value, policy)


# speedup vs baseline: 1.3904x; 1.3904x over previous
import functools

import jax
import jax.numpy as jnp
import numpy as np
from jax import lax
from jax.experimental import pallas as pl
from jax.experimental.pallas import tpu as pltpu
from jax.experimental.pallas import tpu_sc as plsc

K = 3
DIM_FEATURE = 32
B = 4096
Hb = 15
Wb = 15
C = 2
H = Hb - K + 1
P = H * H
NW = 32
BOARDS_PER_W = B // NW
CHUNK = 128
QMAX = 127.0 / 128.0


def _build_unfold_matrix():
    m = np.zeros((C * Hb * Wb, P), dtype=np.float32)
    for c in range(C):
        for di in range(K):
            for dj in range(K):
                w = float(2 ** (c * 9 + di * 3 + dj))
                for i in range(H):
                    for j in range(H):
                        m[c * Hb * Wb + (i + di) * Wb + (j + dj), i * H + j] += w
    return m


_M_CONST = _build_unfold_matrix()
_ZERO_ROWS = np.zeros((CHUNK, DIM_FEATURE // 2), dtype=np.int32)


def _index_body(m_ref, x_ref, o_ref):
    acc = jax.lax.dot_general(
        m_ref[...], x_ref[...], (((0,), (1,)), ((), ())),
        preferred_element_type=jnp.float32)
    o_ref[...] = acc.astype(jnp.int32)


def _compute_indices(board_flat, m):
    blk = 512
    return pl.pallas_call(
        _index_body,
        grid=(B // blk,),
        in_specs=[
            pl.BlockSpec((C * Hb * Wb, P), lambda i: (0, 0)),
            pl.BlockSpec((blk, C * Hb * Wb), lambda i: (i, 0)),
        ],
        out_specs=pl.BlockSpec((P, blk), lambda i: (0, i)),
        out_shape=jax.ShapeDtypeStruct((P, B), jnp.int32),
    )(m, board_flat)


def _quant_body(t_ref, o_ref):
    x = jnp.clip(t_ref[...], -1.0, QMAX)
    u = jnp.round(x * 128.0).astype(jnp.int32) + 128
    lo = jnp.concatenate([u[:, 32 * g:32 * g + 16] for g in range(8)], axis=1)
    hi = jnp.concatenate([u[:, 32 * g + 16:32 * g + 32] for g in range(8)],
                         axis=1)
    o_ref[...] = lo + hi * 65536


def _quantize_table(table):
    v, d = table.shape
    wide = table.reshape(v // 8, 8 * d)
    blk = 2048
    return pl.pallas_call(
        _quant_body,
        grid=(wide.shape[0] // blk,),
        in_specs=[pl.BlockSpec((blk, 8 * d), lambda i: (i, 0))],
        out_specs=pl.BlockSpec((blk, 128), lambda i: (i, 0)),
        out_shape=jax.ShapeDtypeStruct((v // 8, 128), jnp.int32),
    )(wide)


def _sc_gather_sum(qtable, idxpm, zrows):
    mesh = plsc.VectorSubcoreMesh(core_axis_name="c", subcore_axis_name="s")

    @functools.partial(
        pl.kernel,
        out_type=jax.ShapeDtypeStruct((B, DIM_FEATURE // 2), jnp.int32),
        mesh=mesh,
        compiler_params=pltpu.CompilerParams(use_tc_tiling_on_sc=False),
        scratch_types=[
            pltpu.VMEM((P, CHUNK), jnp.int32),
            pltpu.VMEM((CHUNK,), jnp.int32),
            pltpu.VMEM((8, CHUNK, DIM_FEATURE // 2), jnp.int32),
            pltpu.VMEM_SHARED((B // 2, DIM_FEATURE // 2), jnp.int32),
        ] + [pltpu.SemaphoreType.DMA] * 16,
    )
    def sck(qtab_hbm, idx_hbm, zero_hbm, out_hbm,
            idx_v, bidx_v, rows_v, acc, *sems):
        c = lax.axis_index("c")
        s = lax.axis_index("s")
        wid = c * 16 + s
        pltpu.sync_copy(idx_hbm.at[:, pl.ds(wid * CHUNK, CHUNK)], idx_v)
        for k in range(CHUNK // 16):
            bidx_v[pl.ds(k * 16, 16)] = (
                lax.iota(jnp.int32, 16) + (s * CHUNK + k * 16))
        pltpu.sync_copy(zero_hbm, acc.at[pl.ds(s * CHUNK, CHUNK)])

        nbuf = 8
        gs = sems[:nbuf]
        ss = sems[nbuf:]

        def gissue(j, b):
            pltpu.async_copy(qtab_hbm.at[idx_v.at[j]], rows_v.at[b], gs[b])

        def gwait(j, b):
            pltpu.make_async_copy(qtab_hbm.at[idx_v.at[j]],
                                  rows_v.at[b], gs[b]).wait()

        def sissue(b):
            pltpu.async_copy(rows_v.at[b], acc.at[bidx_v], ss[b], add=True)

        def swait(b):
            pltpu.make_async_copy(rows_v.at[b], acc.at[bidx_v], ss[b]).wait()

        for b in range(nbuf):
            gissue(b, b)

        def body(t, carry):
            j = nbuf * t
            for b in range(nbuf):
                gwait(j + b, b)
                sissue(b)
            for b in range(nbuf):
                swait(b)
                gissue(j + nbuf + b, b)
            return carry

        lax.fori_loop(0, (P - nbuf - 1) // nbuf, body, 0)
        for b in range(nbuf):
            gwait(160 + b, b)
            sissue(b)
        swait(0)
        gissue(P - 1, 0)
        for b in range(1, nbuf):
            swait(b)
        gwait(P - 1, 0)
        sissue(0)
        swait(0)
        pltpu.sync_copy(acc.at[pl.ds(s * CHUNK, CHUNK)],
                        out_hbm.at[pl.ds(wid * CHUNK, CHUNK)])

    return sck(qtable, idxpm, zrows)


def _mlp_body(f_ref, w1_ref, b1_ref, w2_ref, b2_ref, w3_ref, b3_ref, o_ref):
    def wq(w):
        return jnp.clip(jnp.round(w * 128.0), -128.0, 127.0) * (1.0 / 128.0)

    def bq(b):
        return jnp.round(b * 16384.0) * (1.0 / 16384.0)

    w = f_ref[...]
    bias = P * 128
    lo = (jnp.bitwise_and(w, 0xFFFF) - bias).astype(jnp.float32)
    hi = (jax.lax.shift_right_logical(w, 16) - bias).astype(jnp.float32)
    v = jnp.concatenate([lo, hi], axis=1) * (1.0 / 128.0)
    v = jnp.clip(v, -1.0, QMAX)
    v = jax.lax.dot_general(v, wq(w1_ref[...]), (((1,), (1,)), ((), ())),
                            preferred_element_type=jnp.float32) + bq(b1_ref[...])
    v = jnp.clip(v, 0.0, QMAX)
    v = jax.lax.dot_general(v, wq(w2_ref[...]), (((1,), (1,)), ((), ())),
                            preferred_element_type=jnp.float32) + bq(b2_ref[...])
    v = jnp.clip(v, 0.0, QMAX)
    v = jax.lax.dot_general(v, wq(w3_ref[...]), (((1,), (1,)), ((), ())),
                            preferred_element_type=jnp.float32) + bq(b3_ref[...])
    o_ref[...] = v


def _mlp(feature, W1, b1, W2, b2, W3, b3):
    return pl.pallas_call(
        _mlp_body,
        out_shape=jax.ShapeDtypeStruct((B, 3), jnp.float32),
    )(feature, W1, b1.reshape(1, -1), W2, b2.reshape(1, -1),
      W3, b3.reshape(1, -1))


def kernel(board_input, hash_features, W1, b1, W2, b2, W3, b3):
    board_flat = board_input.reshape(B, C * Hb * Wb)
    idx = _compute_indices(board_flat, jnp.asarray(_M_CONST))
    qtab = _quantize_table(hash_features)
    qtab = qtab.reshape(2 ** 18, DIM_FEATURE // 2)
    feature = _sc_gather_sum(qtab, idx, jnp.asarray(_ZERO_ROWS))
    value = _mlp(feature, W1, b1, W2, b2, W3, b3)
    policy = jnp.zeros((B, Hb, Wb), dtype=jnp.float32)
    return (value, policy)


# speedup vs baseline: 1.4003x; 1.0071x over previous
import functools

import jax
import jax.numpy as jnp
import numpy as np
from jax import lax
from jax.experimental import pallas as pl
from jax.experimental.pallas import tpu as pltpu
from jax.experimental.pallas import tpu_sc as plsc

K = 3
DIM_FEATURE = 32
B = 4096
Hb = 15
Wb = 15
C = 2
H = Hb - K + 1
P = H * H
NW = 32
BOARDS_PER_W = B // NW
CHUNK = 128
QMAX = 127.0 / 128.0


def _build_unfold_matrix():
    m = np.zeros((C * Hb * Wb, P), dtype=np.float32)
    for c in range(C):
        for di in range(K):
            for dj in range(K):
                w = float(2 ** (c * 9 + di * 3 + dj))
                for i in range(H):
                    for j in range(H):
                        m[c * Hb * Wb + (i + di) * Wb + (j + dj), i * H + j] += w
    return m


_M_CONST = _build_unfold_matrix()
_ZERO_ROWS = np.zeros((CHUNK, DIM_FEATURE // 2), dtype=np.int32)


def _index_body(m_ref, x_ref, o_ref):
    acc = jax.lax.dot_general(
        m_ref[...], x_ref[...], (((0,), (1,)), ((), ())),
        preferred_element_type=jnp.float32)
    o_ref[...] = acc.astype(jnp.int32)


def _compute_indices(board_flat, m):
    blk = 512
    return pl.pallas_call(
        _index_body,
        grid=(B // blk,),
        in_specs=[
            pl.BlockSpec((C * Hb * Wb, P), lambda i: (0, 0)),
            pl.BlockSpec((blk, C * Hb * Wb), lambda i: (i, 0)),
        ],
        out_specs=pl.BlockSpec((P, blk), lambda i: (0, i)),
        out_shape=jax.ShapeDtypeStruct((P, B), jnp.int32),
    )(m, board_flat)


def _quant_body(t_ref, o_ref):
    x = jnp.clip(t_ref[...], -1.0, QMAX)
    u = jnp.round(x * 128.0).astype(jnp.int32) + 128
    lo = jnp.concatenate([u[:, 32 * g:32 * g + 16] for g in range(8)], axis=1)
    hi = jnp.concatenate([u[:, 32 * g + 16:32 * g + 32] for g in range(8)],
                         axis=1)
    o_ref[...] = lo + hi * 65536


def _quantize_table(table):
    v, d = table.shape
    wide = table.reshape(v // 8, 8 * d)
    blk = 2048
    return pl.pallas_call(
        _quant_body,
        grid=(wide.shape[0] // blk,),
        in_specs=[pl.BlockSpec((blk, 8 * d), lambda i: (i, 0))],
        out_specs=pl.BlockSpec((blk, 128), lambda i: (i, 0)),
        out_shape=jax.ShapeDtypeStruct((v // 8, 128), jnp.int32),
    )(wide)


def _sc_gather_sum(qtable, idxpm, zrows):
    mesh = plsc.VectorSubcoreMesh(core_axis_name="c", subcore_axis_name="s")

    @functools.partial(
        pl.kernel,
        out_type=jax.ShapeDtypeStruct((B, DIM_FEATURE // 2), jnp.int32),
        mesh=mesh,
        compiler_params=pltpu.CompilerParams(use_tc_tiling_on_sc=False),
        scratch_types=[
            pltpu.VMEM((P, CHUNK), jnp.int32),
            pltpu.VMEM((CHUNK,), jnp.int32),
            pltpu.VMEM((12, CHUNK, DIM_FEATURE // 2), jnp.int32),
            pltpu.VMEM_SHARED((B // 2, DIM_FEATURE // 2), jnp.int32),
        ] + [pltpu.SemaphoreType.DMA] * 24,
    )
    def sck(qtab_hbm, idx_hbm, zero_hbm, out_hbm,
            idx_v, bidx_v, rows_v, acc, *sems):
        c = lax.axis_index("c")
        s = lax.axis_index("s")
        wid = c * 16 + s
        pltpu.sync_copy(idx_hbm.at[:, pl.ds(wid * CHUNK, CHUNK)], idx_v)
        for k in range(CHUNK // 16):
            bidx_v[pl.ds(k * 16, 16)] = (
                lax.iota(jnp.int32, 16) + (s * CHUNK + k * 16))
        pltpu.sync_copy(zero_hbm, acc.at[pl.ds(s * CHUNK, CHUNK)])

        nbuf = 12
        gs = sems[:nbuf]
        ss = sems[nbuf:]

        def gissue(j, b):
            pltpu.async_copy(qtab_hbm.at[idx_v.at[j]], rows_v.at[b], gs[b])

        def gwait(j, b):
            pltpu.make_async_copy(qtab_hbm.at[idx_v.at[j]],
                                  rows_v.at[b], gs[b]).wait()

        def sissue(b):
            pltpu.async_copy(rows_v.at[b], acc.at[bidx_v], ss[b], add=True)

        def swait(b):
            pltpu.make_async_copy(rows_v.at[b], acc.at[bidx_v], ss[b]).wait()

        iters = (P - nbuf - 1) // nbuf
        assert P == nbuf * iters + nbuf + 1
        for b in range(nbuf):
            gissue(b, b)

        def body(t, carry):
            j = nbuf * t
            for b in range(nbuf):
                gwait(j + b, b)
                sissue(b)
            for b in range(nbuf):
                swait(b)
                gissue(j + nbuf + b, b)
            return carry

        lax.fori_loop(0, iters, body, 0)
        for b in range(nbuf):
            gwait(iters * nbuf + b, b)
            sissue(b)
        swait(0)
        gissue(P - 1, 0)
        for b in range(1, nbuf):
            swait(b)
        gwait(P - 1, 0)
        sissue(0)
        swait(0)
        pltpu.sync_copy(acc.at[pl.ds(s * CHUNK, CHUNK)],
                        out_hbm.at[pl.ds(wid * CHUNK, CHUNK)])

    return sck(qtable, idxpm, zrows)


def _mlp_body(f_ref, w1_ref, b1_ref, w2_ref, b2_ref, w3_ref, b3_ref, o_ref):
    def wq(w):
        return jnp.clip(jnp.round(w * 128.0), -128.0, 127.0) * (1.0 / 128.0)

    def bq(b):
        return jnp.round(b * 16384.0) * (1.0 / 16384.0)

    w = f_ref[...]
    bias = P * 128
    lo = (jnp.bitwise_and(w, 0xFFFF) - bias).astype(jnp.float32)
    hi = (jax.lax.shift_right_logical(w, 16) - bias).astype(jnp.float32)
    v = jnp.concatenate([lo, hi], axis=1) * (1.0 / 128.0)
    v = jnp.clip(v, -1.0, QMAX)
    v = jax.lax.dot_general(v, wq(w1_ref[...]), (((1,), (1,)), ((), ())),
                            preferred_element_type=jnp.float32) + bq(b1_ref[...])
    v = jnp.clip(v, 0.0, QMAX)
    v = jax.lax.dot_general(v, wq(w2_ref[...]), (((1,), (1,)), ((), ())),
                            preferred_element_type=jnp.float32) + bq(b2_ref[...])
    v = jnp.clip(v, 0.0, QMAX)
    v = jax.lax.dot_general(v, wq(w3_ref[...]), (((1,), (1,)), ((), ())),
                            preferred_element_type=jnp.float32) + bq(b3_ref[...])
    o_ref[...] = v


def _mlp(feature, W1, b1, W2, b2, W3, b3):
    return pl.pallas_call(
        _mlp_body,
        out_shape=jax.ShapeDtypeStruct((B, 3), jnp.float32),
    )(feature, W1, b1.reshape(1, -1), W2, b2.reshape(1, -1),
      W3, b3.reshape(1, -1))


def kernel(board_input, hash_features, W1, b1, W2, b2, W3, b3):
    board_flat = board_input.reshape(B, C * Hb * Wb)
    idx = _compute_indices(board_flat, jnp.asarray(_M_CONST))
    qtab = _quantize_table(hash_features)
    qtab = qtab.reshape(2 ** 18, DIM_FEATURE // 2)
    feature = _sc_gather_sum(qtab, idx, jnp.asarray(_ZERO_ROWS))
    value = _mlp(feature, W1, b1, W2, b2, W3, b3)
    policy = jnp.zeros((B, Hb, Wb), dtype=jnp.float32)
    return (value, policy)
